# Initial kernel scaffold; baseline (speedup 1.0000x reference)
import functools

import jax
import jax.numpy as jnp
from jax import lax
from jax.experimental import pallas as pl
from jax.experimental.pallas import tpu as pltpu
from jax.experimental.pallas import tpu_sc as plsc

N = 10000
E = 320000
K = 10
ALPHA = 0.1

NS = 16
L = 16

NPAD = 10240
SLICE = NPAD // NS
EPT = E // NS
EGROUPS = EPT // L
SGROUPS = SLICE // L


def _mlp_body(x_ref, w1_ref, b1_ref, w2_ref, b2_ref, o_ref):
    h1 = jnp.dot(x_ref[...], w1_ref[...], preferred_element_type=jnp.float32)
    h1 = jnp.maximum(h1 + b1_ref[...], 0.0)
    o_ref[...] = (
        jnp.dot(h1, w2_ref[...], preferred_element_type=jnp.float32)
        + b2_ref[...]
    )


def _mlp(x, W1, b1, W2, b2):
    grid = 10
    rows = N // grid
    return pl.pallas_call(
        _mlp_body,
        grid=(grid,),
        in_specs=[
            pl.BlockSpec((rows, 128), lambda i: (i, 0)),
            pl.BlockSpec((128, 256), lambda i: (0, 0)),
            pl.BlockSpec((1, 256), lambda i: (0, 0)),
            pl.BlockSpec((256, 3), lambda i: (0, 0)),
            pl.BlockSpec((1, 3), lambda i: (0, 0)),
        ],
        out_specs=pl.BlockSpec((rows, 3), lambda i: (i, 0)),
        out_shape=jax.ShapeDtypeStruct((N, 3), jnp.float32),
    )(x, W1, b1.reshape(1, 256), W2, b2.reshape(1, 3))


def _fast_rsqrt(x):
    i = plsc.bitcast(x, jnp.int32)
    i = jnp.int32(0x5F3759DF) - lax.shift_right_arithmetic(i, 1)
    y = plsc.bitcast(i, jnp.float32)
    for _ in range(3):
        y = y * (1.5 - 0.5 * x * y * y)
    return y


def _prop_body(h_hbm, src_hbm, dst_hbm, out_hbm,
               h_sh, agg_sh, dis_sh,
               e_src, e_dst, e_nrm,
               h_loc, agg_loc,
               slbuf, snbuf, h0sl, zbuf):
    cid = lax.axis_index("c")
    tid = lax.axis_index("s")
    base_e = pl.multiple_of(tid * EPT, 8)
    nb = pl.multiple_of(tid * SLICE, 8)

    zeros16 = jnp.zeros((L,), jnp.float32)
    ones16 = jnp.ones((L,), jnp.float32)

    def owns(f):
        return (cid == 0) if f < 2 else (cid == 1)

    pltpu.sync_copy(src_hbm.at[pl.ds(base_e, EPT)], e_src)
    pltpu.sync_copy(dst_hbm.at[pl.ds(base_e, EPT)], e_dst)

    def zb_body(i, _):
        zbuf[pl.ds(pl.multiple_of(i * L, 8), L)] = zeros16
        return _
    lax.fori_loop(0, SGROUPS, zb_body, None)

    pltpu.sync_copy(zbuf, dis_sh.at[pl.ds(nb, SLICE)])
    for f in range(3):
        pltpu.sync_copy(zbuf, agg_sh.at[f, pl.ds(nb, SLICE)])

    @pl.when(tid < NS - 1)
    def _():
        for f in range(3):
            pltpu.sync_copy(h_hbm.at[f, pl.ds(nb, SLICE)],
                            h_sh.at[f, pl.ds(nb, SLICE)])
            pltpu.sync_copy(h_hbm.at[f, pl.ds(nb, SLICE)],
                            h0sl.at[pl.ds(f * SLICE, SLICE)])

    @pl.when(tid == NS - 1)
    def _():
        tail = N - (NS - 1) * SLICE
        pad = NPAD - N
        for f in range(3):
            pltpu.sync_copy(h_hbm.at[f, pl.ds((NS - 1) * SLICE, tail)],
                            h_sh.at[f, pl.ds((NS - 1) * SLICE, tail)])
            pltpu.sync_copy(zbuf.at[pl.ds(0, pad)],
                            h_sh.at[f, pl.ds(N, pad)])
            pltpu.sync_copy(h_hbm.at[f, pl.ds((NS - 1) * SLICE, tail)],
                            h0sl.at[pl.ds(f * SLICE, tail)])
            pltpu.sync_copy(zbuf.at[pl.ds(0, pad)],
                            h0sl.at[pl.ds(f * SLICE + tail, pad)])

    def z0_body(i, _):
        agg_loc[pl.ds(pl.multiple_of(i * L, 8), L)] = zeros16
        return _
    lax.fori_loop(0, NPAD // L, z0_body, None)

    def deg_body(g, _):
        d = e_dst[pl.ds(pl.multiple_of(g * L, 8), L)]
        plsc.addupdate_scatter(agg_loc, [d], ones16)
        return _
    lax.fori_loop(0, EGROUPS, deg_body, None)

    pltpu.sync_copy(agg_loc.at[pl.ds(0, NPAD)], dis_sh, add=True)
    plsc.subcore_barrier()

    pltpu.sync_copy(dis_sh.at[pl.ds(nb, SLICE)], slbuf)
    for i in range(SGROUPS):
        sl = pl.ds(i * L, L)
        dv = slbuf[sl] + 1.0
        y = _fast_rsqrt(dv)
        slbuf[sl] = y
        snbuf[sl] = y * y
    pltpu.sync_copy(slbuf, dis_sh.at[pl.ds(nb, SLICE)])
    plsc.subcore_barrier()

    pltpu.sync_copy(dis_sh, agg_loc.at[pl.ds(0, NPAD)])

    def nrm_body(g, _):
        sl = pl.ds(pl.multiple_of(g * L, 8), L)
        s = e_src[sl]
        d = e_dst[sl]
        e_nrm[sl] = plsc.load_gather(agg_loc, [s]) * plsc.load_gather(agg_loc, [d])
        return _
    lax.fori_loop(0, EGROUPS, nrm_body, None)

    def zfull_body(i, _):
        agg_loc[pl.ds(pl.multiple_of(i * L, 8), L)] = zeros16
        return _
    lax.fori_loop(0, 3 * NPAD // L, zfull_body, None)

    def iter_body(k, carry):
        for f in range(3):
            @pl.when(owns(f))
            def _():
                pltpu.sync_copy(h_sh.at[f], h_loc.at[pl.ds(f * NPAD, NPAD)])

        def eb(g, c):
            sl = pl.ds(pl.multiple_of(g * L, 8), L)
            s = e_src[sl]
            d = e_dst[sl]
            n = e_nrm[sl]
            for f in range(3):
                @pl.when(owns(f))
                def _():
                    sf = s + (f * NPAD) if f else s
                    df = d + (f * NPAD) if f else d
                    v = plsc.load_gather(h_loc, [sf])
                    plsc.addupdate_scatter(agg_loc, [df], n * v)
            return c
        lax.fori_loop(0, EGROUPS, eb, None)

        for f in range(3):
            @pl.when(owns(f))
            def _():
                pltpu.sync_copy(agg_loc.at[pl.ds(f * NPAD, NPAD)],
                                agg_sh.at[f], add=True)
        plsc.subcore_barrier()

        for f in range(3):
            @pl.when(owns(f))
            def _():
                pltpu.sync_copy(agg_sh.at[f, pl.ds(nb, SLICE)], slbuf)

                def ub(i, c):
                    b = pl.multiple_of(i * L, 8)
                    hb = pl.multiple_of(f * NPAD + nb + i * L, 8)
                    a = slbuf[pl.ds(b, L)]
                    hc = h_loc[pl.ds(hb, L)]
                    h0 = h0sl[pl.ds(pl.multiple_of(f * SLICE + i * L, 8), L)]
                    hn = (1.0 - ALPHA) * (a + snbuf[pl.ds(b, L)] * hc) + ALPHA * h0
                    slbuf[pl.ds(b, L)] = hn
                    return c
                lax.fori_loop(0, SGROUPS, ub, None)

                pltpu.sync_copy(slbuf, h_sh.at[f, pl.ds(nb, SLICE)])
                pltpu.sync_copy(zbuf, agg_sh.at[f, pl.ds(nb, SLICE)])

                def zrow(i, c):
                    agg_loc[pl.ds(pl.multiple_of(f * NPAD + i * L, 8), L)] = zeros16
                    return c
                lax.fori_loop(0, NPAD // L, zrow, None)
        plsc.subcore_barrier()
        return carry

    lax.fori_loop(0, K, iter_body, None)

    for f in range(3):
        @pl.when(owns(f))
        def _():
            @pl.when(tid < NS - 1)
            def _():
                pltpu.sync_copy(h_sh.at[f, pl.ds(nb, SLICE)],
                                out_hbm.at[f, pl.ds(nb, SLICE)])

            @pl.when(tid == NS - 1)
            def _():
                tail = N - (NS - 1) * SLICE
                pltpu.sync_copy(h_sh.at[f, pl.ds((NS - 1) * SLICE, tail)],
                                out_hbm.at[f, pl.ds((NS - 1) * SLICE, tail)])


@jax.jit
def _propagate(hP, src, dst):
    mesh = plsc.VectorSubcoreMesh(core_axis_name="c", subcore_axis_name="s")
    return pl.kernel(
        _prop_body,
        out_type=jax.ShapeDtypeStruct((3, N), jnp.float32),
        mesh=mesh,
        scratch_types=[
            pltpu.VMEM_SHARED((3, NPAD), jnp.float32),
            pltpu.VMEM_SHARED((3, NPAD), jnp.float32),
            pltpu.VMEM_SHARED((NPAD,), jnp.float32),
            pltpu.VMEM((EPT,), jnp.int32),
            pltpu.VMEM((EPT,), jnp.int32),
            pltpu.VMEM((EPT,), jnp.float32),
            pltpu.VMEM((3 * NPAD,), jnp.float32),
            pltpu.VMEM((3 * NPAD,), jnp.float32),
            pltpu.VMEM((SLICE,), jnp.float32),
            pltpu.VMEM((SLICE,), jnp.float32),
            pltpu.VMEM((3 * SLICE,), jnp.float32),
            pltpu.VMEM((SLICE,), jnp.float32),
        ],
    )(hP, src, dst)


def kernel(x, edge, W1, b1, W2, b2):
    h = _mlp(x, W1, b1, W2, b2)
    hP = h.T
    src = edge[0].astype(jnp.int32)
    dst = edge[1].astype(jnp.int32)
    outP = _propagate(hP, src, dst)
    return outP.T


# speedup vs baseline: 56.6266x; 56.6266x over previous
import jax
import jax.numpy as jnp
from jax import lax
from jax.experimental import pallas as pl
from jax.experimental.pallas import tpu as pltpu
from jax.experimental.pallas import tpu_sc as plsc

N = 10000
E = 320000
K = 10
ALPHA = 0.1

NS = 16
L = 16

NPAD = 10240
SLICE = NPAD // NS
TAIL = N - (NS - 1) * SLICE
PAD = NPAD - N
EPT = E // NS
EGROUPS = EPT // L
SGROUPS = SLICE // L


def _mlp_body(x_ref, w1_ref, b1_ref, w2_ref, b2_ref, o_ref):
    h1 = jnp.dot(x_ref[...], w1_ref[...], preferred_element_type=jnp.float32)
    h1 = jnp.maximum(h1 + b1_ref[...], 0.0)
    o_ref[...] = (
        jnp.dot(h1, w2_ref[...], preferred_element_type=jnp.float32)
        + b2_ref[...]
    )


def _mlp(x, W1, b1, W2, b2):
    grid = 10
    rows = N // grid
    return pl.pallas_call(
        _mlp_body,
        grid=(grid,),
        in_specs=[
            pl.BlockSpec((rows, 128), lambda i: (i, 0)),
            pl.BlockSpec((128, 256), lambda i: (0, 0)),
            pl.BlockSpec((1, 256), lambda i: (0, 0)),
            pl.BlockSpec((256, 3), lambda i: (0, 0)),
            pl.BlockSpec((1, 3), lambda i: (0, 0)),
        ],
        out_specs=pl.BlockSpec((rows, 3), lambda i: (i, 0)),
        out_shape=jax.ShapeDtypeStruct((N, 3), jnp.float32),
    )(x, W1, b1.reshape(1, 256), W2, b2.reshape(1, 3))


def _fast_rsqrt(x):
    i = plsc.bitcast(x, jnp.int32)
    i = jnp.int32(0x5F3759DF) - lax.shift_right_arithmetic(i, 1)
    y = plsc.bitcast(i, jnp.float32)
    for _ in range(3):
        y = y * (1.5 - 0.5 * x * y * y)
    return y


def _prop_body(h_hbm, src_hbm, dst_hbm, out_hbm,
               h_sh, part_sh, dis_sh,
               e_src, e_dst, e_nrm,
               h_loc, agg_loc, tbuf,
               slbuf, snbuf, h0sl, sem):
    cid = lax.axis_index("c")
    tid = lax.axis_index("s")
    base_e = pl.multiple_of(tid * EPT, 8)
    nb = pl.multiple_of(tid * SLICE, 8)
    hbm0 = pl.multiple_of(cid * (2 * N), 8)

    zeros16 = jnp.zeros((L,), jnp.float32)
    ones16 = jnp.ones((L,), jnp.float32)

    pltpu.sync_copy(src_hbm.at[pl.ds(base_e, EPT)], e_src)
    pltpu.sync_copy(dst_hbm.at[pl.ds(base_e, EPT)], e_dst)

    def init_row(lr, fbase):
        @pl.when(tid < NS - 1)
        def _():
            pltpu.sync_copy(h_hbm.at[pl.ds(fbase + nb, SLICE)],
                            h0sl.at[pl.ds(lr * SLICE, SLICE)])

        @pl.when(tid == NS - 1)
        def _():
            base = (NS - 1) * SLICE
            pltpu.sync_copy(h_hbm.at[pl.ds(fbase + base, TAIL)],
                            h0sl.at[pl.ds(lr * SLICE, TAIL)])
            for i in range(PAD // L):
                h0sl[pl.ds(lr * SLICE + TAIL + i * L, L)] = zeros16

        pltpu.sync_copy(h0sl.at[pl.ds(lr * SLICE, SLICE)],
                        h_sh.at[pl.ds(lr * NPAD + nb, SLICE)])

    init_row(0, hbm0)

    @pl.when(cid == 0)
    def _():
        init_row(1, N)

    def zrow0_body(i, c):
        agg_loc[pl.ds(pl.multiple_of(i * L, 8), L)] = zeros16
        return c
    lax.fori_loop(0, NPAD // L, zrow0_body, None)

    def deg_body(g, c):
        d = e_dst[pl.ds(pl.multiple_of(g * L, 8), L)]
        plsc.addupdate_scatter(agg_loc, [d], ones16)
        return c
    lax.fori_loop(0, EGROUPS, deg_body, None)

    part_off = pl.multiple_of(tid * NPAD, 8)
    pltpu.sync_copy(agg_loc.at[pl.ds(0, NPAD)], part_sh.at[pl.ds(part_off, NPAD)])
    plsc.subcore_barrier()

    def accumulate_partials(lr):
        def fetch(t, slot):
            src = part_sh.at[pl.ds(pl.multiple_of((lr * NS + t) * NPAD, 8) + nb,
                                   SLICE)]
            return pltpu.async_copy(src, tbuf.at[pl.ds(slot * SLICE, SLICE)],
                                    sem)

        cp = fetch(0, 0)
        for t in range(NS):
            nxt = fetch(t + 1, (t + 1) % 2) if t + 1 < NS else None
            cp.wait()
            off = (t % 2) * SLICE

            if t == 0:
                def cpy(i, c):
                    sl = pl.ds(pl.multiple_of(i * L, 8), L)
                    slbuf[sl] = tbuf[pl.ds(pl.multiple_of(off + i * L, 8), L)]
                    return c
                lax.fori_loop(0, SGROUPS, cpy, None)
            else:
                def add(i, c, off=off):
                    sl = pl.ds(pl.multiple_of(i * L, 8), L)
                    slbuf[sl] = slbuf[sl] + tbuf[
                        pl.ds(pl.multiple_of(off + i * L, 8), L)]
                    return c
                lax.fori_loop(0, SGROUPS, add, None)
            cp = nxt

    accumulate_partials(0)

    def deg2dis(i, c):
        sl = pl.ds(pl.multiple_of(i * L, 8), L)
        y = _fast_rsqrt(slbuf[sl] + 1.0)
        slbuf[sl] = y
        snbuf[sl] = y * y
        return c
    lax.fori_loop(0, SGROUPS, deg2dis, None)
    pltpu.sync_copy(slbuf, dis_sh.at[pl.ds(nb, SLICE)])
    plsc.subcore_barrier()

    pltpu.sync_copy(dis_sh, agg_loc.at[pl.ds(0, NPAD)])

    def nrm_body(g, c):
        sl = pl.ds(pl.multiple_of(g * L, 8), L)
        s = e_src[sl]
        d = e_dst[sl]
        e_nrm[sl] = plsc.load_gather(agg_loc, [s]) * plsc.load_gather(agg_loc, [d])
        return c
    lax.fori_loop(0, EGROUPS, nrm_body, None)

    def zfull_body(i, c):
        agg_loc[pl.ds(pl.multiple_of(i * L, 8), L)] = zeros16
        return c
    lax.fori_loop(0, 2 * NPAD // L, zfull_body, None)

    def iter_body(k, carry):
        pltpu.sync_copy(h_sh.at[pl.ds(0, NPAD)], h_loc.at[pl.ds(0, NPAD)])

        @pl.when(cid == 0)
        def _():
            pltpu.sync_copy(h_sh.at[pl.ds(NPAD, NPAD)],
                            h_loc.at[pl.ds(NPAD, NPAD)])

        def eb(g, c):
            sl = pl.ds(pl.multiple_of(g * L, 8), L)
            s = e_src[sl]
            d = e_dst[sl]
            n = e_nrm[sl]
            v0 = plsc.load_gather(h_loc, [s])
            plsc.addupdate_scatter(agg_loc, [d], n * v0)

            @pl.when(cid == 0)
            def _():
                s1 = s + NPAD
                d1 = d + NPAD
                v1 = plsc.load_gather(h_loc, [s1])
                plsc.addupdate_scatter(agg_loc, [d1], n * v1)
            return c
        lax.fori_loop(0, EGROUPS, eb, None)

        pltpu.sync_copy(agg_loc.at[pl.ds(0, NPAD)],
                        part_sh.at[pl.ds(part_off, NPAD)])

        @pl.when(cid == 0)
        def _():
            pltpu.sync_copy(agg_loc.at[pl.ds(NPAD, NPAD)],
                            part_sh.at[pl.ds(NS * NPAD + part_off, NPAD)])
        plsc.subcore_barrier()

        def update_row(lr):
            accumulate_partials(lr)

            def ub(i, c):
                sl = pl.ds(pl.multiple_of(i * L, 8), L)
                acc = slbuf[sl]
                hc = h_loc[pl.ds(pl.multiple_of(lr * NPAD + nb + i * L, 8), L)]
                h0 = h0sl[pl.ds(pl.multiple_of(lr * SLICE + i * L, 8), L)]
                hn = (1.0 - ALPHA) * (acc + snbuf[sl] * hc) + ALPHA * h0
                slbuf[sl] = hn
                return c
            lax.fori_loop(0, SGROUPS, ub, None)
            pltpu.sync_copy(slbuf, h_sh.at[pl.ds(lr * NPAD + nb, SLICE)])

            def zrow(i, c):
                agg_loc[pl.ds(pl.multiple_of(lr * NPAD + i * L, 8), L)] = zeros16
                return c
            lax.fori_loop(0, NPAD // L, zrow, None)

        update_row(0)

        @pl.when(cid == 0)
        def _():
            update_row(1)
        plsc.subcore_barrier()
        return carry

    lax.fori_loop(0, K, iter_body, None)

    def out_row(lr, fbase):
        pltpu.sync_copy(h_sh.at[pl.ds(lr * NPAD + nb, SLICE)], slbuf)

        @pl.when(tid < NS - 1)
        def _():
            pltpu.sync_copy(slbuf, out_hbm.at[pl.ds(fbase + nb, SLICE)])

        @pl.when(tid == NS - 1)
        def _():
            base = (NS - 1) * SLICE
            pltpu.sync_copy(slbuf.at[pl.ds(0, TAIL)],
                            out_hbm.at[pl.ds(fbase + base, TAIL)])

    out_row(0, hbm0)

    @pl.when(cid == 0)
    def _():
        out_row(1, N)


@jax.jit
def _propagate(hP, src, dst):
    mesh = plsc.VectorSubcoreMesh(core_axis_name="c", subcore_axis_name="s")
    return pl.kernel(
        _prop_body,
        out_type=jax.ShapeDtypeStruct((3 * N,), jnp.float32),
        mesh=mesh,
        compiler_params=pltpu.CompilerParams(needs_layout_passes=False),
        scratch_types=[
            pltpu.VMEM_SHARED((2 * NPAD,), jnp.float32),
            pltpu.VMEM_SHARED((2 * NS * NPAD,), jnp.float32),
            pltpu.VMEM_SHARED((NPAD,), jnp.float32),
            pltpu.VMEM((EPT,), jnp.int32),
            pltpu.VMEM((EPT,), jnp.int32),
            pltpu.VMEM((EPT,), jnp.float32),
            pltpu.VMEM((2 * NPAD,), jnp.float32),
            pltpu.VMEM((2 * NPAD,), jnp.float32),
            pltpu.VMEM((2 * SLICE,), jnp.float32),
            pltpu.VMEM((SLICE,), jnp.float32),
            pltpu.VMEM((SLICE,), jnp.float32),
            pltpu.VMEM((2 * SLICE,), jnp.float32),
            pltpu.SemaphoreType.DMA,
        ],
    )(hP, src, dst)


def kernel(x, edge, W1, b1, W2, b2):
    h = _mlp(x, W1, b1, W2, b2)
    hP = h.T.reshape(3 * N)
    src = edge[0].astype(jnp.int32)
    dst = edge[1].astype(jnp.int32)
    outP = _propagate(hP, src, dst)
    return outP.reshape(3, N).T


# speedup vs baseline: 60.6091x; 1.0703x over previous
import jax
import jax.numpy as jnp
from jax import lax
from jax.experimental import pallas as pl
from jax.experimental.pallas import tpu as pltpu
from jax.experimental.pallas import tpu_sc as plsc

N = 10000
E = 320000
K = 10
ALPHA = 0.1

NS = 16
L = 16

NPAD = 10240
SLICE = NPAD // NS
TAIL = N - (NS - 1) * SLICE
PAD = NPAD - N
EPT = E // NS
EGROUPS = EPT // L
SGROUPS = SLICE // L

SHIFT = 14
MASK = (1 << SHIFT) - 1


def _mlp_body(x_ref, w1_ref, b1_ref, w2_ref, b2_ref, o_ref):
    h1 = jnp.dot(x_ref[...], w1_ref[...], preferred_element_type=jnp.float32)
    h1 = jnp.maximum(h1 + b1_ref[...], 0.0)
    o_ref[...] = (
        jnp.dot(h1, w2_ref[...], preferred_element_type=jnp.float32)
        + b2_ref[...]
    )


def _mlp(x, W1, b1, W2, b2):
    grid = 10
    rows = N // grid
    return pl.pallas_call(
        _mlp_body,
        grid=(grid,),
        in_specs=[
            pl.BlockSpec((rows, 128), lambda i: (i, 0)),
            pl.BlockSpec((128, 256), lambda i: (0, 0)),
            pl.BlockSpec((1, 256), lambda i: (0, 0)),
            pl.BlockSpec((256, 3), lambda i: (0, 0)),
            pl.BlockSpec((1, 3), lambda i: (0, 0)),
        ],
        out_specs=pl.BlockSpec((rows, 3), lambda i: (i, 0)),
        out_shape=jax.ShapeDtypeStruct((N, 3), jnp.float32),
    )(x, W1, b1.reshape(1, 256), W2, b2.reshape(1, 3))


def _fast_rsqrt(x):
    i = plsc.bitcast(x, jnp.int32)
    i = jnp.int32(0x5F3759DF) - lax.shift_right_arithmetic(i, 1)
    y = plsc.bitcast(i, jnp.float32)
    for _ in range(3):
        y = y * (1.5 - 0.5 * x * y * y)
    return y


def _prop_body(h_hbm, esd_hbm, out_hbm,
               h_sh, part_sh, dis_sh,
               e_sd, e_nrm,
               h_loc, agg_loc, tbuf,
               slbuf, snbuf, h0sl, sem):
    cid = lax.axis_index("c")
    tid = lax.axis_index("s")
    base_e = pl.multiple_of(tid * EPT, 8)
    nb = pl.multiple_of(tid * SLICE, 8)
    pblk = pl.multiple_of(tid * (NS * SLICE), 8)
    hbm0 = pl.multiple_of(cid * (2 * N), 8)

    zeros16 = jnp.zeros((L,), jnp.float32)
    ones16 = jnp.ones((L,), jnp.float32)

    pltpu.sync_copy(esd_hbm.at[pl.ds(base_e, EPT)], e_sd)

    def init_row(lr, fbase):
        @pl.when(tid < NS - 1)
        def _():
            pltpu.sync_copy(h_hbm.at[pl.ds(fbase + nb, SLICE)],
                            h0sl.at[pl.ds(lr * SLICE, SLICE)])

        @pl.when(tid == NS - 1)
        def _():
            base = (NS - 1) * SLICE
            pltpu.sync_copy(h_hbm.at[pl.ds(fbase + base, TAIL)],
                            h0sl.at[pl.ds(lr * SLICE, TAIL)])
            for i in range(PAD // L):
                h0sl[pl.ds(lr * SLICE + TAIL + i * L, L)] = zeros16

        pltpu.sync_copy(h0sl.at[pl.ds(lr * SLICE, SLICE)],
                        h_sh.at[pl.ds(lr * NPAD + nb, SLICE)])

    init_row(0, hbm0)

    @pl.when(cid == 0)
    def _():
        init_row(1, N)

    def zrow0_body(i, c):
        agg_loc[pl.ds(pl.multiple_of(i * L, 8), L)] = zeros16
        return c
    lax.fori_loop(0, NPAD // L, zrow0_body, None)

    def deg_body(g, c):
        sd = e_sd[pl.ds(pl.multiple_of(g * L, 8), L)]
        d = lax.shift_right_logical(sd, SHIFT)
        plsc.addupdate_scatter(agg_loc, [d], ones16)
        return c
    lax.fori_loop(0, EGROUPS, deg_body, None)

    def publish_row(lr):
        cps = []
        for u in range(NS):
            dst = part_sh.at[pl.ds(
                pl.multiple_of((lr * NS + u) * (NS * SLICE), 8) + nb,
                SLICE)]
            cps.append(pltpu.async_copy(
                agg_loc.at[pl.ds(lr * NPAD + u * SLICE, SLICE)], dst, sem))
        for cp in cps:
            cp.wait()

    def fetch_partials(lr):
        src = part_sh.at[pl.ds(
            pl.multiple_of(lr * (NS * NS * SLICE), 8) + pblk, NS * SLICE)]
        pltpu.sync_copy(src, tbuf)

    def sum_partials(i):
        acc = tbuf[pl.ds(pl.multiple_of(i * L, 8), L)]
        for t in range(1, NS):
            acc = acc + tbuf[pl.ds(pl.multiple_of(t * SLICE + i * L, 8), L)]
        return acc

    publish_row(0)
    plsc.subcore_barrier()

    fetch_partials(0)

    def deg2dis(i, c):
        sl = pl.ds(pl.multiple_of(i * L, 8), L)
        y = _fast_rsqrt(sum_partials(i) + 1.0)
        slbuf[sl] = y
        snbuf[sl] = y * y
        return c
    lax.fori_loop(0, SGROUPS, deg2dis, None)
    pltpu.sync_copy(slbuf, dis_sh.at[pl.ds(nb, SLICE)])
    plsc.subcore_barrier()

    pltpu.sync_copy(dis_sh, agg_loc.at[pl.ds(0, NPAD)])

    def nrm_body(g, c):
        sl = pl.ds(pl.multiple_of(g * L, 8), L)
        sd = e_sd[sl]
        s = sd & MASK
        d = lax.shift_right_logical(sd, SHIFT)
        e_nrm[sl] = plsc.load_gather(agg_loc, [s]) * plsc.load_gather(agg_loc, [d])
        return c
    lax.fori_loop(0, EGROUPS, nrm_body, None)

    def zfull_body(i, c):
        agg_loc[pl.ds(pl.multiple_of(i * L, 8), L)] = zeros16
        return c
    lax.fori_loop(0, 2 * NPAD // L, zfull_body, None)

    def iter_body(k, carry):
        pltpu.sync_copy(h_sh.at[pl.ds(0, NPAD)], h_loc.at[pl.ds(0, NPAD)])

        @pl.when(cid == 0)
        def _():
            pltpu.sync_copy(h_sh.at[pl.ds(NPAD, NPAD)],
                            h_loc.at[pl.ds(NPAD, NPAD)])

        def eb(g, c):
            sl = pl.ds(pl.multiple_of(g * L, 8), L)
            sd = e_sd[sl]
            n = e_nrm[sl]
            s = sd & MASK
            d = lax.shift_right_logical(sd, SHIFT)
            v0 = plsc.load_gather(h_loc, [s])
            plsc.addupdate_scatter(agg_loc, [d], n * v0)

            @pl.when(cid == 0)
            def _():
                s1 = s + NPAD
                d1 = d + NPAD
                v1 = plsc.load_gather(h_loc, [s1])
                plsc.addupdate_scatter(agg_loc, [d1], n * v1)
            return c
        lax.fori_loop(0, EGROUPS, eb, None)

        publish_row(0)

        @pl.when(cid == 0)
        def _():
            publish_row(1)
        plsc.subcore_barrier()

        def update_row(lr):
            fetch_partials(lr)

            def ub(i, c):
                sl = pl.ds(pl.multiple_of(i * L, 8), L)
                acc = sum_partials(i)
                hc = h_loc[pl.ds(pl.multiple_of(lr * NPAD + nb + i * L, 8), L)]
                h0 = h0sl[pl.ds(pl.multiple_of(lr * SLICE + i * L, 8), L)]
                hn = (1.0 - ALPHA) * (acc + snbuf[sl] * hc) + ALPHA * h0
                slbuf[sl] = hn
                return c
            lax.fori_loop(0, SGROUPS, ub, None)
            pltpu.sync_copy(slbuf, h_sh.at[pl.ds(lr * NPAD + nb, SLICE)])

            def zrow(i, c):
                agg_loc[pl.ds(pl.multiple_of(lr * NPAD + i * L, 8), L)] = zeros16
                return c
            lax.fori_loop(0, NPAD // L, zrow, None)

        update_row(0)

        @pl.when(cid == 0)
        def _():
            update_row(1)
        plsc.subcore_barrier()
        return carry

    lax.fori_loop(0, K, iter_body, None)

    def out_row(lr, fbase):
        pltpu.sync_copy(h_sh.at[pl.ds(lr * NPAD + nb, SLICE)], slbuf)

        @pl.when(tid < NS - 1)
        def _():
            pltpu.sync_copy(slbuf, out_hbm.at[pl.ds(fbase + nb, SLICE)])

        @pl.when(tid == NS - 1)
        def _():
            base = (NS - 1) * SLICE
            pltpu.sync_copy(slbuf.at[pl.ds(0, TAIL)],
                            out_hbm.at[pl.ds(fbase + base, TAIL)])

    out_row(0, hbm0)

    @pl.when(cid == 0)
    def _():
        out_row(1, N)


@jax.jit
def _propagate(hP, esd):
    mesh = plsc.VectorSubcoreMesh(core_axis_name="c", subcore_axis_name="s")
    return pl.kernel(
        _prop_body,
        out_type=jax.ShapeDtypeStruct((3 * N,), jnp.float32),
        mesh=mesh,
        compiler_params=pltpu.CompilerParams(needs_layout_passes=False),
        scratch_types=[
            pltpu.VMEM_SHARED((2 * NPAD,), jnp.float32),
            pltpu.VMEM_SHARED((2 * NS * NPAD,), jnp.float32),
            pltpu.VMEM_SHARED((NPAD,), jnp.float32),
            pltpu.VMEM((EPT,), jnp.int32),
            pltpu.VMEM((EPT,), jnp.float32),
            pltpu.VMEM((2 * NPAD,), jnp.float32),
            pltpu.VMEM((2 * NPAD,), jnp.float32),
            pltpu.VMEM((NS * SLICE,), jnp.float32),
            pltpu.VMEM((SLICE,), jnp.float32),
            pltpu.VMEM((SLICE,), jnp.float32),
            pltpu.VMEM((2 * SLICE,), jnp.float32),
            pltpu.SemaphoreType.DMA,
        ],
    )(hP, esd)


def kernel(x, edge, W1, b1, W2, b2):
    h = _mlp(x, W1, b1, W2, b2)
    hP = h.T.reshape(3 * N)
    src = edge[0].astype(jnp.int32)
    dst = edge[1].astype(jnp.int32)
    esd = src | (dst << SHIFT)
    outP = _propagate(hP, esd)
    return outP.reshape(3, N).T


# speedup vs baseline: 100.9716x; 1.6659x over previous
import jax
import jax.numpy as jnp
from jax import lax
from jax.experimental import pallas as pl
from jax.experimental.pallas import tpu as pltpu
from jax.experimental.pallas import tpu_sc as plsc

N = 10000
E = 320000
K = 10
ALPHA = 0.1

NS = 16
L = 16

NPAD = 10240
SLICE = NPAD // NS
TAIL = N - (NS - 1) * SLICE
PAD = NPAD - N
EPT = E // NS
EGROUPS = EPT // L
SGROUPS = SLICE // L

SHIFT = 14
MASK = (1 << SHIFT) - 1


def _mlp_body(x_ref, w1_ref, b1_ref, w2_ref, b2_ref, o_ref):
    h1 = jnp.dot(x_ref[...], w1_ref[...], preferred_element_type=jnp.float32)
    h1 = jnp.maximum(h1 + b1_ref[...], 0.0)
    o_ref[...] = (
        jnp.dot(h1, w2_ref[...], preferred_element_type=jnp.float32)
        + b2_ref[...]
    )


def _mlp(x, W1, b1, W2, b2):
    grid = 10
    rows = N // grid
    return pl.pallas_call(
        _mlp_body,
        grid=(grid,),
        in_specs=[
            pl.BlockSpec((rows, 128), lambda i: (i, 0)),
            pl.BlockSpec((128, 256), lambda i: (0, 0)),
            pl.BlockSpec((1, 256), lambda i: (0, 0)),
            pl.BlockSpec((256, 3), lambda i: (0, 0)),
            pl.BlockSpec((1, 3), lambda i: (0, 0)),
        ],
        out_specs=pl.BlockSpec((rows, 3), lambda i: (i, 0)),
        out_shape=jax.ShapeDtypeStruct((N, 3), jnp.float32),
    )(x, W1, b1.reshape(1, 256), W2, b2.reshape(1, 3))


def _fast_rsqrt(x):
    i = plsc.bitcast(x, jnp.int32)
    i = jnp.int32(0x5F3759DF) - lax.shift_right_arithmetic(i, 1)
    y = plsc.bitcast(i, jnp.float32)
    for _ in range(3):
        y = y * (1.5 - 0.5 * x * y * y)
    return y


def _prop_body(h_hbm, esd_hbm, out_hbm,
               h_sh, part_sh, dis_sh,
               e_sd, e_nrm,
               h_loc, agg_loc, tbuf,
               slbuf, snbuf, h0sl, sem):
    cid = lax.axis_index("c")
    tid = lax.axis_index("s")
    base_e = pl.multiple_of(tid * EPT, 8)
    nb = pl.multiple_of(tid * SLICE, 8)
    pblk = pl.multiple_of(tid * (NS * SLICE), 8)
    hbm0 = pl.multiple_of(cid * (2 * N), 8)

    zeros16 = jnp.zeros((L,), jnp.float32)
    ones16 = jnp.ones((L,), jnp.float32)

    pltpu.sync_copy(esd_hbm.at[pl.ds(base_e, EPT)], e_sd)

    def init_row(lr, fbase):
        @pl.when(tid < NS - 1)
        def _():
            pltpu.sync_copy(h_hbm.at[pl.ds(fbase + nb, SLICE)],
                            h0sl.at[pl.ds(lr * SLICE, SLICE)])

        @pl.when(tid == NS - 1)
        def _():
            base = (NS - 1) * SLICE
            pltpu.sync_copy(h_hbm.at[pl.ds(fbase + base, TAIL)],
                            h0sl.at[pl.ds(lr * SLICE, TAIL)])
            for i in range(PAD // L):
                h0sl[pl.ds(lr * SLICE + TAIL + i * L, L)] = zeros16

        pltpu.sync_copy(h0sl.at[pl.ds(lr * SLICE, SLICE)],
                        h_sh.at[pl.ds(lr * NPAD + nb, SLICE)])

    init_row(0, hbm0)

    @pl.when(cid == 0)
    def _():
        init_row(1, N)

    def zrow0_body(i, c):
        agg_loc[pl.ds(pl.multiple_of(i * L, 8), L)] = zeros16
        return c
    lax.fori_loop(0, NPAD // L, zrow0_body, None)

    @plsc.parallel_loop(0, EGROUPS, 1, unroll=4)
    def _(g):
        sd = e_sd[pl.ds(pl.multiple_of(g * L, 8), L)]
        d = lax.shift_right_logical(sd, SHIFT)
        plsc.addupdate_scatter(agg_loc, [d], ones16)

    def publish_row(lr):
        cps = []
        for u in range(NS):
            dst = part_sh.at[pl.ds(
                pl.multiple_of((lr * NS + u) * (NS * SLICE), 8) + nb,
                SLICE)]
            cps.append(pltpu.async_copy(
                agg_loc.at[pl.ds(lr * NPAD + u * SLICE, SLICE)], dst, sem))
        for cp in cps:
            cp.wait()

    def fetch_partials(lr):
        src = part_sh.at[pl.ds(
            pl.multiple_of(lr * (NS * NS * SLICE), 8) + pblk, NS * SLICE)]
        pltpu.sync_copy(src, tbuf)

    def sum_partials(i):
        acc = tbuf[pl.ds(pl.multiple_of(i * L, 8), L)]
        for t in range(1, NS):
            acc = acc + tbuf[pl.ds(pl.multiple_of(t * SLICE + i * L, 8), L)]
        return acc

    publish_row(0)
    plsc.subcore_barrier()

    fetch_partials(0)

    def deg2dis(i, c):
        sl = pl.ds(pl.multiple_of(i * L, 8), L)
        y = _fast_rsqrt(sum_partials(i) + 1.0)
        slbuf[sl] = y
        snbuf[sl] = y * y
        return c
    lax.fori_loop(0, SGROUPS, deg2dis, None)
    pltpu.sync_copy(slbuf, dis_sh.at[pl.ds(nb, SLICE)])
    plsc.subcore_barrier()

    pltpu.sync_copy(dis_sh, agg_loc.at[pl.ds(0, NPAD)])

    @plsc.parallel_loop(0, EGROUPS, 1, unroll=4)
    def _(g):
        sl = pl.ds(pl.multiple_of(g * L, 8), L)
        sd = e_sd[sl]
        s = sd & MASK
        d = lax.shift_right_logical(sd, SHIFT)
        e_nrm[sl] = plsc.load_gather(agg_loc, [s]) * plsc.load_gather(agg_loc, [d])

    def zfull_body(i, c):
        agg_loc[pl.ds(pl.multiple_of(i * L, 8), L)] = zeros16
        return c
    lax.fori_loop(0, 2 * NPAD // L, zfull_body, None)

    def iter_body(k, carry):
        pltpu.sync_copy(h_sh.at[pl.ds(0, NPAD)], h_loc.at[pl.ds(0, NPAD)])

        @pl.when(cid == 0)
        def _():
            pltpu.sync_copy(h_sh.at[pl.ds(NPAD, NPAD)],
                            h_loc.at[pl.ds(NPAD, NPAD)])

        @plsc.parallel_loop(0, EGROUPS, 1, unroll=4)
        def _(g):
            sl = pl.ds(pl.multiple_of(g * L, 8), L)
            sd = e_sd[sl]
            n = e_nrm[sl]
            s = sd & MASK
            d = lax.shift_right_logical(sd, SHIFT)
            v0 = plsc.load_gather(h_loc, [s])
            plsc.addupdate_scatter(agg_loc, [d], n * v0)

            @pl.when(cid == 0)
            def _():
                s1 = s + NPAD
                d1 = d + NPAD
                v1 = plsc.load_gather(h_loc, [s1])
                plsc.addupdate_scatter(agg_loc, [d1], n * v1)

        publish_row(0)

        @pl.when(cid == 0)
        def _():
            publish_row(1)
        plsc.subcore_barrier()

        def update_row(lr):
            fetch_partials(lr)

            def ub(i, c):
                sl = pl.ds(pl.multiple_of(i * L, 8), L)
                acc = sum_partials(i)
                hc = h_loc[pl.ds(pl.multiple_of(lr * NPAD + nb + i * L, 8), L)]
                h0 = h0sl[pl.ds(pl.multiple_of(lr * SLICE + i * L, 8), L)]
                hn = (1.0 - ALPHA) * (acc + snbuf[sl] * hc) + ALPHA * h0
                slbuf[sl] = hn
                return c
            lax.fori_loop(0, SGROUPS, ub, None)
            pltpu.sync_copy(slbuf, h_sh.at[pl.ds(lr * NPAD + nb, SLICE)])

            def zrow(i, c):
                agg_loc[pl.ds(pl.multiple_of(lr * NPAD + i * L, 8), L)] = zeros16
                return c
            lax.fori_loop(0, NPAD // L, zrow, None)

        update_row(0)

        @pl.when(cid == 0)
        def _():
            update_row(1)
        plsc.subcore_barrier()
        return carry

    lax.fori_loop(0, K, iter_body, None)

    def out_row(lr, fbase):
        pltpu.sync_copy(h_sh.at[pl.ds(lr * NPAD + nb, SLICE)], slbuf)

        @pl.when(tid < NS - 1)
        def _():
            pltpu.sync_copy(slbuf, out_hbm.at[pl.ds(fbase + nb, SLICE)])

        @pl.when(tid == NS - 1)
        def _():
            base = (NS - 1) * SLICE
            pltpu.sync_copy(slbuf.at[pl.ds(0, TAIL)],
                            out_hbm.at[pl.ds(fbase + base, TAIL)])

    out_row(0, hbm0)

    @pl.when(cid == 0)
    def _():
        out_row(1, N)


@jax.jit
def _propagate(hP, esd):
    mesh = plsc.VectorSubcoreMesh(core_axis_name="c", subcore_axis_name="s")
    return pl.kernel(
        _prop_body,
        out_type=jax.ShapeDtypeStruct((3 * N,), jnp.float32),
        mesh=mesh,
        compiler_params=pltpu.CompilerParams(needs_layout_passes=False),
        scratch_types=[
            pltpu.VMEM_SHARED((2 * NPAD,), jnp.float32),
            pltpu.VMEM_SHARED((2 * NS * NPAD,), jnp.float32),
            pltpu.VMEM_SHARED((NPAD,), jnp.float32),
            pltpu.VMEM((EPT,), jnp.int32),
            pltpu.VMEM((EPT,), jnp.float32),
            pltpu.VMEM((2 * NPAD,), jnp.float32),
            pltpu.VMEM((2 * NPAD,), jnp.float32),
            pltpu.VMEM((NS * SLICE,), jnp.float32),
            pltpu.VMEM((SLICE,), jnp.float32),
            pltpu.VMEM((SLICE,), jnp.float32),
            pltpu.VMEM((2 * SLICE,), jnp.float32),
            pltpu.SemaphoreType.DMA,
        ],
    )(hP, esd)


def kernel(x, edge, W1, b1, W2, b2):
    h = _mlp(x, W1, b1, W2, b2)
    hP = h.T.reshape(3 * N)
    src = edge[0].astype(jnp.int32)
    dst = edge[1].astype(jnp.int32)
    esd = src | (dst << SHIFT)
    outP = _propagate(hP, esd)
    return outP.reshape(3, N).T
